# baseline (speedup 1.0000x reference)
import functools

import jax
import jax.numpy as jnp
from jax import lax
from jax.experimental import pallas as pl
from jax.experimental.pallas import tpu as pltpu
from jax.experimental.pallas import tpu_sc as plsc

N = 10000
E = 320000
D_IN = 128
HID = 64
LAT = 32

NC = 2
NS = 16
NW = NC * NS
CHUNK = 125
EPW = E // NW
CPW = EPW // CHUNK
RPS = N // NS

DCH = 80
DCPW = EPW // DCH
QR = E * 16 // 128

_MESH = plsc.VectorSubcoreMesh(
    core_axis_name="c", subcore_axis_name="s", num_cores=NC, num_subcores=NS
)
_SC_PARAMS = pltpu.CompilerParams(use_tc_tiling_on_sc=False)


def _wid():
    return lax.axis_index("s") * NC + lax.axis_index("c")


DW = 16


@functools.partial(
    pl.kernel,
    out_type=jax.ShapeDtypeStruct((N, 128), jnp.float32),
    mesh=_MESH,
    compiler_params=_SC_PARAMS,
    scratch_types=[
        pltpu.VMEM((CPW, CHUNK), jnp.int32),
        pltpu.VMEM((CHUNK, DW), jnp.float32),
        pltpu.VMEM_SHARED((N, DW), jnp.float32),
    ],
)
def _deg_call(ed_hbm, z_hbm, one_hbm, out_hbm, edv, ones_v, deg_sh):
    c = lax.axis_index("c")
    s = lax.axis_index("s")
    wid = _wid()
    pltpu.sync_copy(ed_hbm.at[wid], edv)
    pltpu.sync_copy(one_hbm, ones_v)
    pltpu.sync_copy(z_hbm.at[s], deg_sh.at[pl.ds(s * RPS, RPS)])
    plsc.subcore_barrier()

    def body(j, carry):
        pltpu.sync_copy(ones_v, deg_sh.at[edv.at[j]], add=True)
        return carry

    lax.fori_loop(0, CPW, body, 0)
    plsc.subcore_barrier()
    pltpu.sync_copy(
        deg_sh.at[pl.ds(s * RPS, RPS)],
        out_hbm.at[pl.ds(s * RPS, RPS), pl.ds(c * DW, DW)],
    )


@functools.partial(
    pl.kernel,
    out_type=jax.ShapeDtypeStruct((N, 128), jnp.float32),
    mesh=_MESH,
    compiler_params=_SC_PARAMS,
    scratch_types=[
        pltpu.VMEM((CPW, CHUNK), jnp.int32),
        pltpu.VMEM((CPW, CHUNK), jnp.int32),
        pltpu.VMEM((CHUNK, HID), jnp.float32),
        pltpu.VMEM((CHUNK, HID), jnp.float32),
        pltpu.SemaphoreType.DMA,
        pltpu.SemaphoreType.DMA,
        pltpu.VMEM_SHARED((N, HID), jnp.float32),
    ],
)
def _mp_call(
    g_hbm, es_hbm, ed_hbm, z_hbm, out_hbm, esv, edv, rows0, rows1, semA, semB, acc_sh
):
    c = lax.axis_index("c")
    s = lax.axis_index("s")
    wid = _wid()
    pltpu.sync_copy(es_hbm.at[wid], esv)
    pltpu.sync_copy(ed_hbm.at[wid], edv)
    pltpu.sync_copy(z_hbm.at[s], acc_sh.at[pl.ds(s * RPS, RPS)])
    plsc.subcore_barrier()

    pltpu.async_copy(g_hbm.at[esv.at[0]], rows0, semA)

    def body(j, carry):
        e0 = 2 * j
        e1 = e0 + 1
        pltpu.async_copy(g_hbm.at[esv.at[e1]], rows1, semB)
        pltpu.make_async_copy(g_hbm.at[esv.at[e0]], rows0, semA).wait()
        pltpu.sync_copy(rows0, acc_sh.at[edv.at[e0]], add=True)

        @pl.when(e0 + 2 < CPW)
        def _():
            pltpu.async_copy(g_hbm.at[esv.at[e0 + 2]], rows0, semA)

        pltpu.make_async_copy(g_hbm.at[esv.at[e1]], rows1, semB).wait()
        pltpu.sync_copy(rows1, acc_sh.at[edv.at[e1]], add=True)
        return carry

    lax.fori_loop(0, CPW // 2, body, 0)
    plsc.subcore_barrier()
    pltpu.sync_copy(
        acc_sh.at[pl.ds(s * RPS, RPS)],
        out_hbm.at[pl.ds(s * RPS, RPS), pl.ds(c * HID, HID)],
    )


@functools.partial(
    pl.kernel,
    out_type=jax.ShapeDtypeStruct((QR, 128), jnp.float32),
    mesh=_MESH,
    compiler_params=_SC_PARAMS,
    scratch_types=[
        pltpu.VMEM((DCPW, DCH), jnp.int32),
        pltpu.VMEM((DCPW, DCH), jnp.int32),
        pltpu.VMEM((DCH, LAT), jnp.float32),
        pltpu.VMEM((DCH, LAT), jnp.float32),
        pltpu.VMEM((DCH, LAT), jnp.float32),
        pltpu.VMEM((DCH, LAT), jnp.float32),
        pltpu.VMEM((DCH // 8, 128), jnp.float32),
        pltpu.VMEM((DCH // 8, 128), jnp.float32),
        pltpu.SemaphoreType.DMA,
        pltpu.SemaphoreType.DMA,
        pltpu.SemaphoreType.DMA,
        pltpu.SemaphoreType.DMA,
        pltpu.SemaphoreType.DMA,
        pltpu.SemaphoreType.DMA,
    ],
)
def _dec_call(
    mu_hbm, src_hbm, dst_hbm, q_hbm,
    siv, div, arow0, brow0, arow1, brow1, qv0, qv1, sa0, sb0, sa1, sb1, sq0, sq1,
):
    wid = _wid()
    pltpu.sync_copy(src_hbm.at[wid], siv)
    pltpu.sync_copy(dst_hbm.at[wid], div)

    RPC = DCH // 8

    def compute(arow, brow, qv, sq, j):
        @pl.when(j >= 2)
        def _():
            pltpu.make_async_copy(
                qv, q_hbm.at[pl.ds(wid * EPW // 8 + (j - 2) * RPC, RPC)], sq
            ).wait()

        def inner(i, carry2):
            for k in range(8):
                e = i * 8 + k
                a0 = arow[e, pl.ds(0, 16)]
                a1 = arow[e, pl.ds(16, 16)]
                b0 = brow[e, pl.ds(0, 16)]
                b1 = brow[e, pl.ds(16, 16)]
                qv[i, pl.ds(k * 16, 16)] = a0 * b0 + a1 * b1
            return carry2

        lax.fori_loop(0, RPC, inner, 0)
        pltpu.async_copy(qv, q_hbm.at[pl.ds(wid * EPW // 8 + j * RPC, RPC)], sq)

    pltpu.async_copy(mu_hbm.at[siv.at[0]], arow0, sa0)
    pltpu.async_copy(mu_hbm.at[div.at[0]], brow0, sb0)

    def body(j, carry):
        e0 = 2 * j
        e1 = e0 + 1
        pltpu.async_copy(mu_hbm.at[siv.at[e1]], arow1, sa1)
        pltpu.async_copy(mu_hbm.at[div.at[e1]], brow1, sb1)
        pltpu.make_async_copy(mu_hbm.at[siv.at[e0]], arow0, sa0).wait()
        pltpu.make_async_copy(mu_hbm.at[div.at[e0]], brow0, sb0).wait()
        compute(arow0, brow0, qv0, sq0, e0)

        @pl.when(e0 + 2 < DCPW)
        def _():
            pltpu.async_copy(mu_hbm.at[siv.at[e0 + 2]], arow0, sa0)
            pltpu.async_copy(mu_hbm.at[div.at[e0 + 2]], brow0, sb0)

        pltpu.make_async_copy(mu_hbm.at[siv.at[e1]], arow1, sa1).wait()
        pltpu.make_async_copy(mu_hbm.at[div.at[e1]], brow1, sb1).wait()
        compute(arow1, brow1, qv1, sq1, e1)
        return carry

    lax.fori_loop(0, DCPW // 2, body, 0)
    eL = DCPW - 1
    pltpu.make_async_copy(mu_hbm.at[siv.at[eL]], arow0, sa0).wait()
    pltpu.make_async_copy(mu_hbm.at[div.at[eL]], brow0, sb0).wait()
    compute(arow0, brow0, qv0, sq0, eL)
    pltpu.make_async_copy(
        qv1, q_hbm.at[pl.ds(wid * EPW // 8 + (eL - 1) * (DCH // 8), DCH // 8)], sq1
    ).wait()
    pltpu.make_async_copy(
        qv0, q_hbm.at[pl.ds(wid * EPW // 8 + eL * (DCH // 8), DCH // 8)], sq0
    ).wait()


BR = 2000


def _mm1_body(x_ref, w1_ref, h_ref):
    h_ref[...] = jnp.dot(x_ref[...], w1_ref[...], preferred_element_type=jnp.float32)


def _mm1(x, W1):
    return pl.pallas_call(
        _mm1_body,
        grid=(N // BR,),
        in_specs=[
            pl.BlockSpec((BR, D_IN), lambda i: (i, 0)),
            pl.BlockSpec((D_IN, HID), lambda i: (0, 0)),
        ],
        out_specs=pl.BlockSpec((BR, HID), lambda i: (i, 0)),
        out_shape=jax.ShapeDtypeStruct((N, HID), jnp.float32),
    )(x, W1)


def _dinv_of(dp):
    deg = dp[:, 0:1] + dp[:, DW : DW + 1] + 1.0
    return lax.rsqrt(deg)


def _k1_body(dp_ref, h_ref, g1_ref):
    g1_ref[...] = h_ref[...] * _dinv_of(dp_ref[...])


def _k1(degp, h):
    return pl.pallas_call(
        _k1_body,
        grid=(N // BR,),
        in_specs=[
            pl.BlockSpec((BR, 128), lambda i: (i, 0)),
            pl.BlockSpec((BR, HID), lambda i: (i, 0)),
        ],
        out_specs=pl.BlockSpec((BR, HID), lambda i: (i, 0)),
        out_shape=jax.ShapeDtypeStruct((N, HID), jnp.float32),
    )(degp, h)


def _k2_body(a_ref, g_ref, dp_ref, b_ref, w_ref, g2_ref):
    A = a_ref[:, 0:HID] + a_ref[:, HID : 2 * HID]
    dinv = _dinv_of(dp_ref[...])
    h = jnp.maximum(dinv * (A + g_ref[...]) + b_ref[...], 0.0)
    t = jnp.dot(h, w_ref[...], preferred_element_type=jnp.float32)
    g2_ref[...] = t * dinv


def _k2(a, g, degp, b, W):
    return pl.pallas_call(
        _k2_body,
        grid=(N // BR,),
        in_specs=[
            pl.BlockSpec((BR, 128), lambda i: (i, 0)),
            pl.BlockSpec((BR, HID), lambda i: (i, 0)),
            pl.BlockSpec((BR, 128), lambda i: (i, 0)),
            pl.BlockSpec((HID,), lambda i: (0,)),
            pl.BlockSpec((HID, HID), lambda i: (0, 0)),
        ],
        out_specs=pl.BlockSpec((BR, HID), lambda i: (i, 0)),
        out_shape=jax.ShapeDtypeStruct((N, HID), jnp.float32),
    )(a, g, degp, b, W)


def _k3_body(a_ref, g_ref, dp_ref, b_ref, wmu_ref, bmu_ref, mu_ref):
    A = a_ref[:, 0:HID] + a_ref[:, HID : 2 * HID]
    dinv = _dinv_of(dp_ref[...])
    h = jnp.maximum(dinv * (A + g_ref[...]) + b_ref[...], 0.0)
    mu_ref[...] = (
        jnp.dot(h, wmu_ref[...], preferred_element_type=jnp.float32) + bmu_ref[...]
    )


def _k3(a, g, degp, b, Wmu, bmu):
    return pl.pallas_call(
        _k3_body,
        grid=(N // BR,),
        in_specs=[
            pl.BlockSpec((BR, 128), lambda i: (i, 0)),
            pl.BlockSpec((BR, HID), lambda i: (i, 0)),
            pl.BlockSpec((BR, 128), lambda i: (i, 0)),
            pl.BlockSpec((HID,), lambda i: (0,)),
            pl.BlockSpec((HID, LAT), lambda i: (0, 0)),
            pl.BlockSpec((LAT,), lambda i: (0,)),
        ],
        out_specs=pl.BlockSpec((BR, LAT), lambda i: (i, 0)),
        out_shape=jax.ShapeDtypeStruct((N, LAT), jnp.float32),
    )(a, g, degp, b, Wmu, bmu)


BR4 = 8000


def _k4_body(q_ref, s_ref, out_ref):
    z = lax.dot_general(
        s_ref[...],
        q_ref[...],
        dimension_numbers=(((0,), (1,)), ((), ())),
        preferred_element_type=jnp.float32,
    )
    out_ref[...] = 1.0 / (1.0 + jnp.exp(-z))


def _k4(q2, sel):
    return pl.pallas_call(
        _k4_body,
        out_shape=jax.ShapeDtypeStruct((8, QR), jnp.float32),
    )(q2, sel)


def kernel(x, edge_index, src, dst, W1, b1, W2, b2, Wmu, bmu):
    es2 = edge_index[0].reshape(NW, CPW, CHUNK)
    ed2 = edge_index[1].reshape(NW, CPW, CHUNK)
    src2 = src.reshape(NW, DCPW, DCH)
    dst2 = dst.reshape(NW, DCPW, DCH)
    zeros_nd = jnp.zeros((NS, RPS, DW), jnp.float32)
    zeros_nh = jnp.zeros((NS, RPS, HID), jnp.float32)
    ones_c = jnp.ones((CHUNK, DW), jnp.float32)
    sel = (jnp.arange(128)[:, None] // 16 == jnp.arange(8)[None, :]).astype(
        jnp.float32
    )

    h0 = _mm1(x, W1)
    degp = _deg_call(ed2, zeros_nd, ones_c)
    g1 = _k1(degp, h0)
    a1 = _mp_call(g1, es2, ed2, zeros_nh)
    g2 = _k2(a1, g1, degp, b1, W2)
    a2 = _mp_call(g2, es2, ed2, zeros_nh)
    mu = _k3(a2, g2, degp, b2, Wmu, bmu)
    q = _dec_call(mu, src2, dst2)
    out = _k4(q, sel)
    return out.T.reshape(E)


# speedup vs baseline: 1.0308x; 1.0308x over previous
import functools

import jax
import jax.numpy as jnp
from jax import lax
from jax.experimental import pallas as pl
from jax.experimental.pallas import tpu as pltpu
from jax.experimental.pallas import tpu_sc as plsc

N = 10000
E = 320000
D_IN = 128
HID = 64
LAT = 32

NC = 2
NS = 16
NW = NC * NS
CHUNK = 125
EPW = E // NW
CPW = EPW // CHUNK
RPS = N // NS

DCH = 80
DCPW = EPW // DCH
QR = E * 16 // 128

_MESH = plsc.VectorSubcoreMesh(
    core_axis_name="c", subcore_axis_name="s", num_cores=NC, num_subcores=NS
)
_SC_PARAMS = pltpu.CompilerParams(use_tc_tiling_on_sc=False)


def _wid():
    return lax.axis_index("s") * NC + lax.axis_index("c")


DW = 16


@functools.partial(
    pl.kernel,
    out_type=jax.ShapeDtypeStruct((N, 128), jnp.float32),
    mesh=_MESH,
    compiler_params=_SC_PARAMS,
    scratch_types=[
        pltpu.VMEM((CPW, CHUNK), jnp.int32),
        pltpu.VMEM((CHUNK, DW), jnp.float32),
        pltpu.VMEM_SHARED((N, DW), jnp.float32),
    ],
)
def _deg_call(ed_hbm, z_hbm, one_hbm, out_hbm, edv, ones_v, deg_sh):
    c = lax.axis_index("c")
    s = lax.axis_index("s")
    wid = _wid()
    pltpu.sync_copy(ed_hbm.at[wid], edv)
    pltpu.sync_copy(one_hbm, ones_v)
    pltpu.sync_copy(z_hbm.at[s], deg_sh.at[pl.ds(s * RPS, RPS)])
    plsc.subcore_barrier()

    def body(j, carry):
        pltpu.sync_copy(ones_v, deg_sh.at[edv.at[j]], add=True)
        return carry

    lax.fori_loop(0, CPW, body, 0)
    plsc.subcore_barrier()
    pltpu.sync_copy(
        deg_sh.at[pl.ds(s * RPS, RPS)],
        out_hbm.at[pl.ds(s * RPS, RPS), pl.ds(c * DW, DW)],
    )


@functools.partial(
    pl.kernel,
    out_type=jax.ShapeDtypeStruct((N, 128), jnp.float32),
    mesh=_MESH,
    compiler_params=_SC_PARAMS,
    scratch_types=[
        pltpu.VMEM((CPW, CHUNK), jnp.int32),
        pltpu.VMEM((CPW, CHUNK), jnp.int32),
        pltpu.VMEM((CHUNK, HID), jnp.float32),
        pltpu.VMEM((CHUNK, HID), jnp.float32),
        pltpu.SemaphoreType.DMA,
        pltpu.SemaphoreType.DMA,
        pltpu.VMEM_SHARED((N, HID), jnp.float32),
    ],
)
def _mp_call(
    g_hbm, es_hbm, ed_hbm, z_hbm, out_hbm, esv, edv, rows0, rows1, semA, semB, acc_sh
):
    c = lax.axis_index("c")
    s = lax.axis_index("s")
    wid = _wid()
    pltpu.sync_copy(es_hbm.at[wid], esv)
    pltpu.sync_copy(ed_hbm.at[wid], edv)
    pltpu.sync_copy(z_hbm.at[s], acc_sh.at[pl.ds(s * RPS, RPS)])
    plsc.subcore_barrier()

    pltpu.async_copy(g_hbm.at[esv.at[0]], rows0, semA)

    def body(j, carry):
        e0 = 2 * j
        e1 = e0 + 1
        pltpu.async_copy(g_hbm.at[esv.at[e1]], rows1, semB)
        pltpu.make_async_copy(g_hbm.at[esv.at[e0]], rows0, semA).wait()
        pltpu.sync_copy(rows0, acc_sh.at[edv.at[e0]], add=True)

        @pl.when(e0 + 2 < CPW)
        def _():
            pltpu.async_copy(g_hbm.at[esv.at[e0 + 2]], rows0, semA)

        pltpu.make_async_copy(g_hbm.at[esv.at[e1]], rows1, semB).wait()
        pltpu.sync_copy(rows1, acc_sh.at[edv.at[e1]], add=True)
        return carry

    lax.fori_loop(0, CPW // 2, body, 0)
    plsc.subcore_barrier()
    pltpu.sync_copy(
        acc_sh.at[pl.ds(s * RPS, RPS)],
        out_hbm.at[pl.ds(s * RPS, RPS), pl.ds(c * HID, HID)],
    )


@functools.partial(
    pl.kernel,
    out_type=jax.ShapeDtypeStruct((QR, 128), jnp.float32),
    mesh=_MESH,
    compiler_params=_SC_PARAMS,
    scratch_types=[
        pltpu.VMEM((DCPW, DCH), jnp.int32),
        pltpu.VMEM((DCPW, DCH), jnp.int32),
        pltpu.VMEM((DCH, LAT), jnp.float32),
        pltpu.VMEM((DCH, LAT), jnp.float32),
        pltpu.VMEM((DCH, LAT), jnp.float32),
        pltpu.VMEM((DCH, LAT), jnp.float32),
        pltpu.VMEM((DCH // 8, 128), jnp.float32),
        pltpu.VMEM((DCH // 8, 128), jnp.float32),
        pltpu.SemaphoreType.DMA,
        pltpu.SemaphoreType.DMA,
        pltpu.SemaphoreType.DMA,
        pltpu.SemaphoreType.DMA,
        pltpu.SemaphoreType.DMA,
        pltpu.SemaphoreType.DMA,
    ],
)
def _dec_call(
    mu_hbm, src_hbm, dst_hbm, q_hbm,
    siv, div, arow0, brow0, arow1, brow1, qv0, qv1, sa0, sb0, sa1, sb1, sq0, sq1,
):
    wid = _wid()
    pltpu.sync_copy(src_hbm.at[wid], siv)
    pltpu.sync_copy(dst_hbm.at[wid], div)

    RPC = DCH // 8

    def compute(arow, brow, qv, sq, j):
        @pl.when(j >= 2)
        def _():
            pltpu.make_async_copy(
                qv, q_hbm.at[pl.ds(wid * EPW // 8 + (j - 2) * RPC, RPC)], sq
            ).wait()

        def inner(i, carry2):
            for k in range(8):
                e = i * 8 + k
                a0 = arow[e, pl.ds(0, 16)]
                a1 = arow[e, pl.ds(16, 16)]
                b0 = brow[e, pl.ds(0, 16)]
                b1 = brow[e, pl.ds(16, 16)]
                qv[i, pl.ds(k * 16, 16)] = a0 * b0 + a1 * b1
            return carry2

        lax.fori_loop(0, RPC, inner, 0)
        pltpu.async_copy(qv, q_hbm.at[pl.ds(wid * EPW // 8 + j * RPC, RPC)], sq)

    pltpu.async_copy(mu_hbm.at[siv.at[0]], arow0, sa0)
    pltpu.async_copy(mu_hbm.at[div.at[0]], brow0, sb0)

    def body(j, carry):
        e0 = 2 * j
        e1 = e0 + 1
        pltpu.async_copy(mu_hbm.at[siv.at[e1]], arow1, sa1)
        pltpu.async_copy(mu_hbm.at[div.at[e1]], brow1, sb1)
        pltpu.make_async_copy(mu_hbm.at[siv.at[e0]], arow0, sa0).wait()
        pltpu.make_async_copy(mu_hbm.at[div.at[e0]], brow0, sb0).wait()
        compute(arow0, brow0, qv0, sq0, e0)

        @pl.when(e0 + 2 < DCPW)
        def _():
            pltpu.async_copy(mu_hbm.at[siv.at[e0 + 2]], arow0, sa0)
            pltpu.async_copy(mu_hbm.at[div.at[e0 + 2]], brow0, sb0)

        pltpu.make_async_copy(mu_hbm.at[siv.at[e1]], arow1, sa1).wait()
        pltpu.make_async_copy(mu_hbm.at[div.at[e1]], brow1, sb1).wait()
        compute(arow1, brow1, qv1, sq1, e1)
        return carry

    lax.fori_loop(0, DCPW // 2, body, 0)
    eL = DCPW - 1
    pltpu.make_async_copy(mu_hbm.at[siv.at[eL]], arow0, sa0).wait()
    pltpu.make_async_copy(mu_hbm.at[div.at[eL]], brow0, sb0).wait()
    compute(arow0, brow0, qv0, sq0, eL)
    pltpu.make_async_copy(
        qv1, q_hbm.at[pl.ds(wid * EPW // 8 + (eL - 1) * (DCH // 8), DCH // 8)], sq1
    ).wait()
    pltpu.make_async_copy(
        qv0, q_hbm.at[pl.ds(wid * EPW // 8 + eL * (DCH // 8), DCH // 8)], sq0
    ).wait()


BR = 2000


def _mm1_body(x_ref, w1_ref, h_ref):
    h_ref[...] = jnp.dot(x_ref[...], w1_ref[...], preferred_element_type=jnp.float32)


def _mm1(x, W1):
    return pl.pallas_call(
        _mm1_body,
        grid=(N // BR,),
        in_specs=[
            pl.BlockSpec((BR, D_IN), lambda i: (i, 0)),
            pl.BlockSpec((D_IN, HID), lambda i: (0, 0)),
        ],
        out_specs=pl.BlockSpec((BR, HID), lambda i: (i, 0)),
        out_shape=jax.ShapeDtypeStruct((N, HID), jnp.float32),
    )(x, W1)


def _dinv_of(dp):
    deg = dp[:, 0:1] + dp[:, DW : DW + 1] + 1.0
    return lax.rsqrt(deg)


def _k1_body(dp_ref, h_ref, g1_ref):
    g1_ref[...] = h_ref[...] * _dinv_of(dp_ref[...])


def _k1(degp, h):
    return pl.pallas_call(
        _k1_body,
        grid=(N // BR,),
        in_specs=[
            pl.BlockSpec((BR, 128), lambda i: (i, 0)),
            pl.BlockSpec((BR, HID), lambda i: (i, 0)),
        ],
        out_specs=pl.BlockSpec((BR, HID), lambda i: (i, 0)),
        out_shape=jax.ShapeDtypeStruct((N, HID), jnp.float32),
    )(degp, h)


def _k2_body(a_ref, g_ref, dp_ref, b_ref, w_ref, g2_ref):
    A = a_ref[:, 0:HID] + a_ref[:, HID : 2 * HID]
    dinv = _dinv_of(dp_ref[...])
    h = jnp.maximum(dinv * (A + g_ref[...]) + b_ref[...], 0.0)
    t = jnp.dot(h, w_ref[...], preferred_element_type=jnp.float32)
    g2_ref[...] = t * dinv


def _k2(a, g, degp, b, W):
    return pl.pallas_call(
        _k2_body,
        grid=(N // BR,),
        in_specs=[
            pl.BlockSpec((BR, 128), lambda i: (i, 0)),
            pl.BlockSpec((BR, HID), lambda i: (i, 0)),
            pl.BlockSpec((BR, 128), lambda i: (i, 0)),
            pl.BlockSpec((HID,), lambda i: (0,)),
            pl.BlockSpec((HID, HID), lambda i: (0, 0)),
        ],
        out_specs=pl.BlockSpec((BR, HID), lambda i: (i, 0)),
        out_shape=jax.ShapeDtypeStruct((N, HID), jnp.float32),
    )(a, g, degp, b, W)


def _k3_body(a_ref, g_ref, dp_ref, b_ref, wmu_ref, bmu_ref, mu_ref):
    A = a_ref[:, 0:HID] + a_ref[:, HID : 2 * HID]
    dinv = _dinv_of(dp_ref[...])
    h = jnp.maximum(dinv * (A + g_ref[...]) + b_ref[...], 0.0)
    mu_ref[...] = (
        jnp.dot(h, wmu_ref[...], preferred_element_type=jnp.float32) + bmu_ref[...]
    )


def _k3(a, g, degp, b, Wmu, bmu):
    return pl.pallas_call(
        _k3_body,
        grid=(N // BR,),
        in_specs=[
            pl.BlockSpec((BR, 128), lambda i: (i, 0)),
            pl.BlockSpec((BR, HID), lambda i: (i, 0)),
            pl.BlockSpec((BR, 128), lambda i: (i, 0)),
            pl.BlockSpec((HID,), lambda i: (0,)),
            pl.BlockSpec((HID, LAT), lambda i: (0, 0)),
            pl.BlockSpec((LAT,), lambda i: (0,)),
        ],
        out_specs=pl.BlockSpec((BR, LAT), lambda i: (i, 0)),
        out_shape=jax.ShapeDtypeStruct((N, LAT), jnp.float32),
    )(a, g, degp, b, Wmu, bmu)


BR4 = 8000


def _k4_body(q_ref, s_ref, out_ref):
    z = lax.dot_general(
        s_ref[...],
        q_ref[...],
        dimension_numbers=(((0,), (1,)), ((), ())),
        preferred_element_type=jnp.float32,
    )
    out_ref[...] = 1.0 / (1.0 + jnp.exp(-z))


def _k4(q2, sel):
    return pl.pallas_call(
        _k4_body,
        out_shape=jax.ShapeDtypeStruct((8, QR), jnp.float32),
    )(q2, sel)


def kernel(x, edge_index, src, dst, W1, b1, W2, b2, Wmu, bmu):
    es2 = edge_index[0].reshape(NW, CPW, CHUNK)
    ed2 = edge_index[1].reshape(NW, CPW, CHUNK)
    src2 = src.reshape(8, NW, DCPW, DCH // 8).transpose(1, 2, 3, 0).reshape(
        NW, DCPW, DCH
    )
    dst2 = dst.reshape(8, NW, DCPW, DCH // 8).transpose(1, 2, 3, 0).reshape(
        NW, DCPW, DCH
    )
    zeros_nd = jnp.zeros((NS, RPS, DW), jnp.float32)
    zeros_nh = jnp.zeros((NS, RPS, HID), jnp.float32)
    ones_c = jnp.ones((CHUNK, DW), jnp.float32)
    sel = (jnp.arange(128)[:, None] // 16 == jnp.arange(8)[None, :]).astype(
        jnp.float32
    )

    h0 = _mm1(x, W1)
    degp = _deg_call(ed2, zeros_nd, ones_c)
    g1 = _k1(degp, h0)
    a1 = _mp_call(g1, es2, ed2, zeros_nh)
    g2 = _k2(a1, g1, degp, b1, W2)
    a2 = _mp_call(g2, es2, ed2, zeros_nh)
    mu = _k3(a2, g2, degp, b2, Wmu, bmu)
    q = _dec_call(mu, src2, dst2)
    return _k4(q, sel).reshape(E)


# speedup vs baseline: 1.0556x; 1.0241x over previous
import functools

import jax
import jax.numpy as jnp
from jax import lax
from jax.experimental import pallas as pl
from jax.experimental.pallas import tpu as pltpu
from jax.experimental.pallas import tpu_sc as plsc

N = 10000
E = 320000
D_IN = 128
HID = 64
LAT = 32

NC = 2
NS = 16
NW = NC * NS
CHUNK = 125
EPW = E // NW
CPW = EPW // CHUNK
RPS = N // NS

DCH = 80
DCPW = EPW // DCH
QR = E * 16 // 128

_MESH = plsc.VectorSubcoreMesh(
    core_axis_name="c", subcore_axis_name="s", num_cores=NC, num_subcores=NS
)
_SC_PARAMS = pltpu.CompilerParams(use_tc_tiling_on_sc=False)


def _wid():
    return lax.axis_index("s") * NC + lax.axis_index("c")


DW = 16


@functools.partial(
    pl.kernel,
    out_type=jax.ShapeDtypeStruct((N, 128), jnp.float32),
    mesh=_MESH,
    compiler_params=_SC_PARAMS,
    scratch_types=[
        pltpu.VMEM((CPW, CHUNK), jnp.int32),
        pltpu.VMEM((CHUNK, DW), jnp.float32),
        pltpu.VMEM_SHARED((N, DW), jnp.float32),
    ],
)
def _deg_call(ei_hbm, z_hbm, one_hbm, out_hbm, edv, ones_v, deg_sh):
    c = lax.axis_index("c")
    s = lax.axis_index("s")
    wid = _wid()
    pltpu.sync_copy(ei_hbm.at[1, wid], edv)
    pltpu.sync_copy(one_hbm, ones_v)
    pltpu.sync_copy(z_hbm.at[s], deg_sh.at[pl.ds(s * RPS, RPS)])
    plsc.subcore_barrier()

    def body(j, carry):
        pltpu.sync_copy(ones_v, deg_sh.at[edv.at[j]], add=True)
        return carry

    lax.fori_loop(0, CPW, body, 0)
    plsc.subcore_barrier()
    pltpu.sync_copy(
        deg_sh.at[pl.ds(s * RPS, RPS)],
        out_hbm.at[pl.ds(s * RPS, RPS), pl.ds(c * DW, DW)],
    )


@functools.partial(
    pl.kernel,
    out_type=jax.ShapeDtypeStruct((N, 128), jnp.float32),
    mesh=_MESH,
    compiler_params=_SC_PARAMS,
    scratch_types=[
        pltpu.VMEM((CPW, CHUNK), jnp.int32),
        pltpu.VMEM((CPW, CHUNK), jnp.int32),
        pltpu.VMEM((CHUNK, HID), jnp.float32),
        pltpu.VMEM((CHUNK, HID), jnp.float32),
        pltpu.SemaphoreType.DMA,
        pltpu.SemaphoreType.DMA,
        pltpu.VMEM_SHARED((N, HID), jnp.float32),
    ],
)
def _mp_call(
    g_hbm, ei_hbm, z_hbm, out_hbm, esv, edv, rows0, rows1, semA, semB, acc_sh
):
    c = lax.axis_index("c")
    s = lax.axis_index("s")
    wid = _wid()
    pltpu.sync_copy(ei_hbm.at[0, wid], esv)
    pltpu.sync_copy(ei_hbm.at[1, wid], edv)
    pltpu.sync_copy(z_hbm.at[s], acc_sh.at[pl.ds(s * RPS, RPS)])
    plsc.subcore_barrier()

    pltpu.async_copy(g_hbm.at[esv.at[0]], rows0, semA)

    def body(j, carry):
        e0 = 2 * j
        e1 = e0 + 1
        pltpu.async_copy(g_hbm.at[esv.at[e1]], rows1, semB)
        pltpu.make_async_copy(g_hbm.at[esv.at[e0]], rows0, semA).wait()
        pltpu.sync_copy(rows0, acc_sh.at[edv.at[e0]], add=True)

        @pl.when(e0 + 2 < CPW)
        def _():
            pltpu.async_copy(g_hbm.at[esv.at[e0 + 2]], rows0, semA)

        pltpu.make_async_copy(g_hbm.at[esv.at[e1]], rows1, semB).wait()
        pltpu.sync_copy(rows1, acc_sh.at[edv.at[e1]], add=True)
        return carry

    lax.fori_loop(0, CPW // 2, body, 0)
    plsc.subcore_barrier()
    pltpu.sync_copy(
        acc_sh.at[pl.ds(s * RPS, RPS)],
        out_hbm.at[pl.ds(s * RPS, RPS), pl.ds(c * HID, HID)],
    )


@functools.partial(
    pl.kernel,
    out_type=jax.ShapeDtypeStruct((QR, 128), jnp.float32),
    mesh=_MESH,
    compiler_params=_SC_PARAMS,
    scratch_types=[
        pltpu.VMEM((DCPW, DCH), jnp.int32),
        pltpu.VMEM((DCPW, DCH), jnp.int32),
        pltpu.VMEM((DCH, LAT), jnp.float32),
        pltpu.VMEM((DCH, LAT), jnp.float32),
        pltpu.VMEM((DCH, LAT), jnp.float32),
        pltpu.VMEM((DCH, LAT), jnp.float32),
        pltpu.VMEM((DCH // 8, 128), jnp.float32),
        pltpu.VMEM((DCH // 8, 128), jnp.float32),
        pltpu.SemaphoreType.DMA,
        pltpu.SemaphoreType.DMA,
        pltpu.SemaphoreType.DMA,
        pltpu.SemaphoreType.DMA,
        pltpu.SemaphoreType.DMA,
        pltpu.SemaphoreType.DMA,
    ],
)
def _dec_call(
    mu_hbm, src_hbm, dst_hbm, q_hbm,
    siv, div, arow0, brow0, arow1, brow1, qv0, qv1, sa0, sb0, sa1, sb1, sq0, sq1,
):
    wid = _wid()
    pltpu.sync_copy(src_hbm.at[wid], siv)
    pltpu.sync_copy(dst_hbm.at[wid], div)

    RPC = DCH // 8

    def compute(arow, brow, qv, sq, j):
        @pl.when(j >= 2)
        def _():
            pltpu.make_async_copy(
                qv, q_hbm.at[pl.ds(wid * EPW // 8 + (j - 2) * RPC, RPC)], sq
            ).wait()

        def inner(i, carry2):
            for k in range(8):
                e = i * 8 + k
                a0 = arow[e, pl.ds(0, 16)]
                a1 = arow[e, pl.ds(16, 16)]
                b0 = brow[e, pl.ds(0, 16)]
                b1 = brow[e, pl.ds(16, 16)]
                qv[i, pl.ds(k * 16, 16)] = a0 * b0 + a1 * b1
            return carry2

        lax.fori_loop(0, RPC, inner, 0)
        pltpu.async_copy(qv, q_hbm.at[pl.ds(wid * EPW // 8 + j * RPC, RPC)], sq)

    pltpu.async_copy(mu_hbm.at[siv.at[0]], arow0, sa0)
    pltpu.async_copy(mu_hbm.at[div.at[0]], brow0, sb0)

    def body(j, carry):
        e0 = 2 * j
        e1 = e0 + 1
        pltpu.async_copy(mu_hbm.at[siv.at[e1]], arow1, sa1)
        pltpu.async_copy(mu_hbm.at[div.at[e1]], brow1, sb1)
        pltpu.make_async_copy(mu_hbm.at[siv.at[e0]], arow0, sa0).wait()
        pltpu.make_async_copy(mu_hbm.at[div.at[e0]], brow0, sb0).wait()
        compute(arow0, brow0, qv0, sq0, e0)

        @pl.when(e0 + 2 < DCPW)
        def _():
            pltpu.async_copy(mu_hbm.at[siv.at[e0 + 2]], arow0, sa0)
            pltpu.async_copy(mu_hbm.at[div.at[e0 + 2]], brow0, sb0)

        pltpu.make_async_copy(mu_hbm.at[siv.at[e1]], arow1, sa1).wait()
        pltpu.make_async_copy(mu_hbm.at[div.at[e1]], brow1, sb1).wait()
        compute(arow1, brow1, qv1, sq1, e1)
        return carry

    lax.fori_loop(0, DCPW // 2, body, 0)
    eL = DCPW - 1
    pltpu.make_async_copy(mu_hbm.at[siv.at[eL]], arow0, sa0).wait()
    pltpu.make_async_copy(mu_hbm.at[div.at[eL]], brow0, sb0).wait()
    compute(arow0, brow0, qv0, sq0, eL)
    pltpu.make_async_copy(
        qv1, q_hbm.at[pl.ds(wid * EPW // 8 + (eL - 1) * (DCH // 8), DCH // 8)], sq1
    ).wait()
    pltpu.make_async_copy(
        qv0, q_hbm.at[pl.ds(wid * EPW // 8 + eL * (DCH // 8), DCH // 8)], sq0
    ).wait()


BR = 2000


def _mm1_body(x_ref, w1_ref, h_ref):
    h_ref[...] = jnp.dot(x_ref[...], w1_ref[...], preferred_element_type=jnp.float32)


def _mm1(x, W1):
    return pl.pallas_call(
        _mm1_body,
        grid=(N // BR,),
        in_specs=[
            pl.BlockSpec((BR, D_IN), lambda i: (i, 0)),
            pl.BlockSpec((D_IN, HID), lambda i: (0, 0)),
        ],
        out_specs=pl.BlockSpec((BR, HID), lambda i: (i, 0)),
        out_shape=jax.ShapeDtypeStruct((N, HID), jnp.float32),
    )(x, W1)


def _dinv_of(dp):
    deg = dp[:, 0:1] + dp[:, DW : DW + 1] + 1.0
    return lax.rsqrt(deg)


def _k1_body(dp_ref, h_ref, g1_ref):
    g1_ref[...] = h_ref[...] * _dinv_of(dp_ref[...])


def _k1(degp, h):
    return pl.pallas_call(
        _k1_body,
        grid=(N // BR,),
        in_specs=[
            pl.BlockSpec((BR, 128), lambda i: (i, 0)),
            pl.BlockSpec((BR, HID), lambda i: (i, 0)),
        ],
        out_specs=pl.BlockSpec((BR, HID), lambda i: (i, 0)),
        out_shape=jax.ShapeDtypeStruct((N, HID), jnp.float32),
    )(degp, h)


def _k2_body(a_ref, g_ref, dp_ref, b_ref, w_ref, g2_ref):
    A = a_ref[:, 0:HID] + a_ref[:, HID : 2 * HID]
    dinv = _dinv_of(dp_ref[...])
    h = jnp.maximum(dinv * (A + g_ref[...]) + b_ref[...], 0.0)
    t = jnp.dot(h, w_ref[...], preferred_element_type=jnp.float32)
    g2_ref[...] = t * dinv


def _k2(a, g, degp, b, W):
    return pl.pallas_call(
        _k2_body,
        grid=(N // BR,),
        in_specs=[
            pl.BlockSpec((BR, 128), lambda i: (i, 0)),
            pl.BlockSpec((BR, HID), lambda i: (i, 0)),
            pl.BlockSpec((BR, 128), lambda i: (i, 0)),
            pl.BlockSpec((HID,), lambda i: (0,)),
            pl.BlockSpec((HID, HID), lambda i: (0, 0)),
        ],
        out_specs=pl.BlockSpec((BR, HID), lambda i: (i, 0)),
        out_shape=jax.ShapeDtypeStruct((N, HID), jnp.float32),
    )(a, g, degp, b, W)


def _k3_body(a_ref, g_ref, dp_ref, b_ref, wmu_ref, bmu_ref, mu_ref):
    A = a_ref[:, 0:HID] + a_ref[:, HID : 2 * HID]
    dinv = _dinv_of(dp_ref[...])
    h = jnp.maximum(dinv * (A + g_ref[...]) + b_ref[...], 0.0)
    mu_ref[...] = (
        jnp.dot(h, wmu_ref[...], preferred_element_type=jnp.float32) + bmu_ref[...]
    )


def _k3(a, g, degp, b, Wmu, bmu):
    return pl.pallas_call(
        _k3_body,
        grid=(N // BR,),
        in_specs=[
            pl.BlockSpec((BR, 128), lambda i: (i, 0)),
            pl.BlockSpec((BR, HID), lambda i: (i, 0)),
            pl.BlockSpec((BR, 128), lambda i: (i, 0)),
            pl.BlockSpec((HID,), lambda i: (0,)),
            pl.BlockSpec((HID, LAT), lambda i: (0, 0)),
            pl.BlockSpec((LAT,), lambda i: (0,)),
        ],
        out_specs=pl.BlockSpec((BR, LAT), lambda i: (i, 0)),
        out_shape=jax.ShapeDtypeStruct((N, LAT), jnp.float32),
    )(a, g, degp, b, Wmu, bmu)


BR4 = 8000


def _k4_body(q_ref, s_ref, out_ref):
    z = lax.dot_general(
        s_ref[...],
        q_ref[...],
        dimension_numbers=(((0,), (1,)), ((), ())),
        preferred_element_type=jnp.float32,
    )
    out_ref[...] = 1.0 / (1.0 + jnp.exp(-z))


def _k4(q2, sel):
    return pl.pallas_call(
        _k4_body,
        out_shape=jax.ShapeDtypeStruct((8, QR), jnp.float32),
    )(q2, sel)


def kernel(x, edge_index, src, dst, W1, b1, W2, b2, Wmu, bmu):
    ei2 = edge_index.reshape(2, NW, CPW, CHUNK)
    src2 = src.reshape(8, NW, DCPW, DCH // 8).transpose(1, 2, 3, 0).reshape(
        NW, DCPW, DCH
    )
    dst2 = dst.reshape(8, NW, DCPW, DCH // 8).transpose(1, 2, 3, 0).reshape(
        NW, DCPW, DCH
    )
    zeros_nd = jnp.zeros((NS, RPS, DW), jnp.float32)
    zeros_nh = jnp.zeros((NS, RPS, HID), jnp.float32)
    ones_c = jnp.ones((CHUNK, DW), jnp.float32)
    sel = (jnp.arange(128)[:, None] // 16 == jnp.arange(8)[None, :]).astype(
        jnp.float32
    )

    h0 = _mm1(x, W1)
    degp = _deg_call(ei2, zeros_nd, ones_c)
    g1 = _k1(degp, h0)
    a1 = _mp_call(g1, ei2, zeros_nh)
    g2 = _k2(a1, g1, degp, b1, W2)
    a2 = _mp_call(g2, ei2, zeros_nh)
    mu = _k3(a2, g2, degp, b2, Wmu, bmu)
    q = _dec_call(mu, src2, dst2)
    return _k4(q, sel).reshape(E)


# speedup vs baseline: 1.0640x; 1.0080x over previous
import functools

import jax
import jax.numpy as jnp
from jax import lax
from jax.experimental import pallas as pl
from jax.experimental.pallas import tpu as pltpu
from jax.experimental.pallas import tpu_sc as plsc

N = 10000
E = 320000
D_IN = 128
HID = 64
LAT = 32

NC = 2
NS = 16
NW = NC * NS
CHUNK = 125
EPW = E // NW
CPW = EPW // CHUNK
RPS = N // NS

DCH = 80
DCPW = EPW // DCH
QR = E * 16 // 128

_MESH = plsc.VectorSubcoreMesh(
    core_axis_name="c", subcore_axis_name="s", num_cores=NC, num_subcores=NS
)
_SC_PARAMS = pltpu.CompilerParams(use_tc_tiling_on_sc=False)


def _wid():
    return lax.axis_index("s") * NC + lax.axis_index("c")


DW = 16


@functools.partial(
    pl.kernel,
    out_type=jax.ShapeDtypeStruct((N, 128), jnp.float32),
    mesh=_MESH,
    compiler_params=_SC_PARAMS,
    scratch_types=[
        pltpu.VMEM((CPW, CHUNK), jnp.int32),
        pltpu.VMEM((CHUNK, DW), jnp.float32),
        pltpu.VMEM_SHARED((N, DW), jnp.float32),
    ],
)
def _deg_call(ei_hbm, z_hbm, one_hbm, out_hbm, edv, ones_v, deg_sh):
    c = lax.axis_index("c")
    s = lax.axis_index("s")
    wid = _wid()
    pltpu.sync_copy(ei_hbm.at[1, wid], edv)
    pltpu.sync_copy(one_hbm, ones_v)
    pltpu.sync_copy(z_hbm.at[s], deg_sh.at[pl.ds(s * RPS, RPS)])
    plsc.subcore_barrier()

    def body(j, carry):
        pltpu.sync_copy(ones_v, deg_sh.at[edv.at[j]], add=True)
        return carry

    lax.fori_loop(0, CPW, body, 0)
    plsc.subcore_barrier()
    pltpu.sync_copy(
        deg_sh.at[pl.ds(s * RPS, RPS)],
        out_hbm.at[pl.ds(s * RPS, RPS), pl.ds(c * DW, DW)],
    )


@functools.partial(
    pl.kernel,
    out_type=jax.ShapeDtypeStruct((N, 128), jnp.float32),
    mesh=_MESH,
    compiler_params=_SC_PARAMS,
    scratch_types=[
        pltpu.VMEM((CPW, CHUNK), jnp.int32),
        pltpu.VMEM((CPW, CHUNK), jnp.int32),
        pltpu.VMEM((CHUNK, HID), jnp.float32),
        pltpu.VMEM((CHUNK, HID), jnp.float32),
        pltpu.SemaphoreType.DMA,
        pltpu.SemaphoreType.DMA,
        pltpu.VMEM_SHARED((N, HID), jnp.float32),
    ],
)
def _mp_call(
    g_hbm, ei_hbm, z_hbm, out_hbm, esv, edv, rows0, rows1, semA, semB, acc_sh
):
    c = lax.axis_index("c")
    s = lax.axis_index("s")
    wid = _wid()
    pltpu.sync_copy(ei_hbm.at[0, wid], esv)
    pltpu.sync_copy(ei_hbm.at[1, wid], edv)
    pltpu.sync_copy(z_hbm.at[s], acc_sh.at[pl.ds(s * RPS, RPS)])
    plsc.subcore_barrier()

    pltpu.async_copy(g_hbm.at[esv.at[0]], rows0, semA)

    def body(j, carry):
        e0 = 2 * j
        e1 = e0 + 1
        pltpu.async_copy(g_hbm.at[esv.at[e1]], rows1, semB)
        pltpu.make_async_copy(g_hbm.at[esv.at[e0]], rows0, semA).wait()
        pltpu.sync_copy(rows0, acc_sh.at[edv.at[e0]], add=True)

        @pl.when(e0 + 2 < CPW)
        def _():
            pltpu.async_copy(g_hbm.at[esv.at[e0 + 2]], rows0, semA)

        pltpu.make_async_copy(g_hbm.at[esv.at[e1]], rows1, semB).wait()
        pltpu.sync_copy(rows1, acc_sh.at[edv.at[e1]], add=True)
        return carry

    lax.fori_loop(0, CPW // 2, body, 0)
    plsc.subcore_barrier()
    pltpu.sync_copy(
        acc_sh.at[pl.ds(s * RPS, RPS)],
        out_hbm.at[pl.ds(s * RPS, RPS), pl.ds(c * HID, HID)],
    )


@functools.partial(
    pl.kernel,
    out_type=jax.ShapeDtypeStruct((QR, 128), jnp.float32),
    mesh=_MESH,
    compiler_params=_SC_PARAMS,
    scratch_types=[
        pltpu.VMEM((DCPW, DCH), jnp.int32),
        pltpu.VMEM((DCPW, DCH), jnp.int32),
        pltpu.VMEM((DCH, LAT), jnp.float32),
        pltpu.VMEM((DCH, LAT), jnp.float32),
        pltpu.VMEM((DCH, LAT), jnp.float32),
        pltpu.VMEM((DCH, LAT), jnp.float32),
        pltpu.VMEM((DCH // 8, 128), jnp.float32),
        pltpu.VMEM((DCH // 8, 128), jnp.float32),
        pltpu.SemaphoreType.DMA,
        pltpu.SemaphoreType.DMA,
        pltpu.SemaphoreType.DMA,
        pltpu.SemaphoreType.DMA,
        pltpu.SemaphoreType.DMA,
        pltpu.SemaphoreType.DMA,
    ],
)
def _dec_call(
    mu_hbm, src_hbm, dst_hbm, q_hbm,
    siv, div, arow0, brow0, arow1, brow1, qv0, qv1, sa0, sb0, sa1, sb1, sq0, sq1,
):
    wid = _wid()
    pltpu.sync_copy(src_hbm.at[wid], siv)
    pltpu.sync_copy(dst_hbm.at[wid], div)

    RPC = DCH // 8

    def compute(arow, brow, qv, sq, j):
        @pl.when(j >= 2)
        def _():
            pltpu.make_async_copy(
                qv, q_hbm.at[pl.ds(wid * EPW // 8 + (j - 2) * RPC, RPC)], sq
            ).wait()

        for i in range(RPC):
            for k in range(8):
                e = i * 8 + k
                a0 = arow[e, pl.ds(0, 16)]
                a1 = arow[e, pl.ds(16, 16)]
                b0 = brow[e, pl.ds(0, 16)]
                b1 = brow[e, pl.ds(16, 16)]
                qv[i, pl.ds(k * 16, 16)] = a0 * b0 + a1 * b1
        pltpu.async_copy(qv, q_hbm.at[pl.ds(wid * EPW // 8 + j * RPC, RPC)], sq)

    pltpu.async_copy(mu_hbm.at[siv.at[0]], arow0, sa0)
    pltpu.async_copy(mu_hbm.at[div.at[0]], brow0, sb0)

    def body(j, carry):
        e0 = 2 * j
        e1 = e0 + 1
        pltpu.async_copy(mu_hbm.at[siv.at[e1]], arow1, sa1)
        pltpu.async_copy(mu_hbm.at[div.at[e1]], brow1, sb1)
        pltpu.make_async_copy(mu_hbm.at[siv.at[e0]], arow0, sa0).wait()
        pltpu.make_async_copy(mu_hbm.at[div.at[e0]], brow0, sb0).wait()
        compute(arow0, brow0, qv0, sq0, e0)

        @pl.when(e0 + 2 < DCPW)
        def _():
            pltpu.async_copy(mu_hbm.at[siv.at[e0 + 2]], arow0, sa0)
            pltpu.async_copy(mu_hbm.at[div.at[e0 + 2]], brow0, sb0)

        pltpu.make_async_copy(mu_hbm.at[siv.at[e1]], arow1, sa1).wait()
        pltpu.make_async_copy(mu_hbm.at[div.at[e1]], brow1, sb1).wait()
        compute(arow1, brow1, qv1, sq1, e1)
        return carry

    lax.fori_loop(0, DCPW // 2, body, 0)
    eL = DCPW - 1
    pltpu.make_async_copy(mu_hbm.at[siv.at[eL]], arow0, sa0).wait()
    pltpu.make_async_copy(mu_hbm.at[div.at[eL]], brow0, sb0).wait()
    compute(arow0, brow0, qv0, sq0, eL)
    pltpu.make_async_copy(
        qv1, q_hbm.at[pl.ds(wid * EPW // 8 + (eL - 1) * (DCH // 8), DCH // 8)], sq1
    ).wait()
    pltpu.make_async_copy(
        qv0, q_hbm.at[pl.ds(wid * EPW // 8 + eL * (DCH // 8), DCH // 8)], sq0
    ).wait()


BR = 2000


def _mm1_body(x_ref, w1_ref, h_ref):
    h_ref[...] = jnp.dot(x_ref[...], w1_ref[...], preferred_element_type=jnp.float32)


def _mm1(x, W1):
    return pl.pallas_call(
        _mm1_body,
        grid=(N // BR,),
        in_specs=[
            pl.BlockSpec((BR, D_IN), lambda i: (i, 0)),
            pl.BlockSpec((D_IN, HID), lambda i: (0, 0)),
        ],
        out_specs=pl.BlockSpec((BR, HID), lambda i: (i, 0)),
        out_shape=jax.ShapeDtypeStruct((N, HID), jnp.float32),
    )(x, W1)


def _dinv_of(dp):
    deg = dp[:, 0:1] + dp[:, DW : DW + 1] + 1.0
    return lax.rsqrt(deg)


def _k1_body(dp_ref, h_ref, g1_ref):
    g1_ref[...] = h_ref[...] * _dinv_of(dp_ref[...])


def _k1(degp, h):
    return pl.pallas_call(
        _k1_body,
        grid=(N // BR,),
        in_specs=[
            pl.BlockSpec((BR, 128), lambda i: (i, 0)),
            pl.BlockSpec((BR, HID), lambda i: (i, 0)),
        ],
        out_specs=pl.BlockSpec((BR, HID), lambda i: (i, 0)),
        out_shape=jax.ShapeDtypeStruct((N, HID), jnp.float32),
    )(degp, h)


def _k2_body(a_ref, g_ref, dp_ref, b_ref, w_ref, g2_ref):
    A = a_ref[:, 0:HID] + a_ref[:, HID : 2 * HID]
    dinv = _dinv_of(dp_ref[...])
    h = jnp.maximum(dinv * (A + g_ref[...]) + b_ref[...], 0.0)
    t = jnp.dot(h, w_ref[...], preferred_element_type=jnp.float32)
    g2_ref[...] = t * dinv


def _k2(a, g, degp, b, W):
    return pl.pallas_call(
        _k2_body,
        grid=(N // BR,),
        in_specs=[
            pl.BlockSpec((BR, 128), lambda i: (i, 0)),
            pl.BlockSpec((BR, HID), lambda i: (i, 0)),
            pl.BlockSpec((BR, 128), lambda i: (i, 0)),
            pl.BlockSpec((HID,), lambda i: (0,)),
            pl.BlockSpec((HID, HID), lambda i: (0, 0)),
        ],
        out_specs=pl.BlockSpec((BR, HID), lambda i: (i, 0)),
        out_shape=jax.ShapeDtypeStruct((N, HID), jnp.float32),
    )(a, g, degp, b, W)


def _k3_body(a_ref, g_ref, dp_ref, b_ref, wmu_ref, bmu_ref, mu_ref):
    A = a_ref[:, 0:HID] + a_ref[:, HID : 2 * HID]
    dinv = _dinv_of(dp_ref[...])
    h = jnp.maximum(dinv * (A + g_ref[...]) + b_ref[...], 0.0)
    mu_ref[...] = (
        jnp.dot(h, wmu_ref[...], preferred_element_type=jnp.float32) + bmu_ref[...]
    )


def _k3(a, g, degp, b, Wmu, bmu):
    return pl.pallas_call(
        _k3_body,
        grid=(N // BR,),
        in_specs=[
            pl.BlockSpec((BR, 128), lambda i: (i, 0)),
            pl.BlockSpec((BR, HID), lambda i: (i, 0)),
            pl.BlockSpec((BR, 128), lambda i: (i, 0)),
            pl.BlockSpec((HID,), lambda i: (0,)),
            pl.BlockSpec((HID, LAT), lambda i: (0, 0)),
            pl.BlockSpec((LAT,), lambda i: (0,)),
        ],
        out_specs=pl.BlockSpec((BR, LAT), lambda i: (i, 0)),
        out_shape=jax.ShapeDtypeStruct((N, LAT), jnp.float32),
    )(a, g, degp, b, Wmu, bmu)


BR4 = 8000


def _k4_body(q_ref, s_ref, out_ref):
    z = lax.dot_general(
        s_ref[...],
        q_ref[...],
        dimension_numbers=(((0,), (1,)), ((), ())),
        preferred_element_type=jnp.float32,
    )
    out_ref[...] = 1.0 / (1.0 + jnp.exp(-z))


def _k4(q2, sel):
    return pl.pallas_call(
        _k4_body,
        out_shape=jax.ShapeDtypeStruct((8, QR), jnp.float32),
    )(q2, sel)


def kernel(x, edge_index, src, dst, W1, b1, W2, b2, Wmu, bmu):
    ei2 = edge_index.reshape(2, NW, CPW, CHUNK)
    src2 = src.reshape(8, NW, DCPW, DCH // 8).transpose(1, 2, 3, 0).reshape(
        NW, DCPW, DCH
    )
    dst2 = dst.reshape(8, NW, DCPW, DCH // 8).transpose(1, 2, 3, 0).reshape(
        NW, DCPW, DCH
    )
    zeros_nd = jnp.zeros((NS, RPS, DW), jnp.float32)
    zeros_nh = jnp.zeros((NS, RPS, HID), jnp.float32)
    ones_c = jnp.ones((CHUNK, DW), jnp.float32)
    sel = (jnp.arange(128)[:, None] // 16 == jnp.arange(8)[None, :]).astype(
        jnp.float32
    )

    h0 = _mm1(x, W1)
    degp = _deg_call(ei2, zeros_nd, ones_c)
    g1 = _k1(degp, h0)
    a1 = _mp_call(g1, ei2, zeros_nh)
    g2 = _k2(a1, g1, degp, b1, W2)
    a2 = _mp_call(g2, ei2, zeros_nh)
    mu = _k3(a2, g2, degp, b2, Wmu, bmu)
    q = _dec_call(mu, src2, dst2)
    return _k4(q, sel).reshape(E)
